# baseline (speedup 1.0000x reference)
import functools

import jax
import jax.numpy as jnp
from jax import lax
from jax.experimental import pallas as pl
from jax.experimental.pallas import tpu as pltpu
from jax.experimental.pallas import tpu_sc as plsc

N_NODES = 10000
N_EDGES = 320000
D = 128
CW = 16
EPS = 1e-5

NUM_CORES = 2
NUM_SUBCORES = 16
NUM_WORKERS = NUM_CORES * NUM_SUBCORES
EDGES_PER_WORKER = N_EDGES // NUM_WORKERS
CHUNK = 40
NUM_CHUNKS = EDGES_PER_WORKER // CHUNK
NBUF = 6
NBUFI = 8
ROWS_PER_SUBCORE = N_NODES // NUM_SUBCORES


def _sc_agg_kernel(x_hbm, eidx_hbm, agg_hbm, cnt_hbm, agg_sp, cnt_sp,
                   idx_ring, rows_v, ones_v, zb_v, sem_g, sem_s, sem_c,
                   sem_x):
    c = lax.axis_index("c")
    s = lax.axis_index("s")
    wid = s * NUM_CORES + c
    chunk_base = wid * NUM_CHUNKS

    def start_idx(i, bi):
        e0 = (chunk_base + i) * CHUNK
        pltpu.async_copy(eidx_hbm.at[0, pl.ds(e0, CHUNK)], idx_ring.at[bi, 0],
                         sem_x.at[bi])
        pltpu.async_copy(eidx_hbm.at[1, pl.ds(e0, CHUNK)], idx_ring.at[bi, 1],
                         sem_x.at[bi])

    def wait_idx(i, bi):
        e0 = (chunk_base + i) * CHUNK
        pltpu.make_async_copy(eidx_hbm.at[0, pl.ds(e0, CHUNK)],
                              idx_ring.at[bi, 0], sem_x.at[bi]).wait()
        pltpu.make_async_copy(eidx_hbm.at[1, pl.ds(e0, CHUNK)],
                              idx_ring.at[bi, 1], sem_x.at[bi]).wait()

    def start_gather(i, b):
        pltpu.async_copy(x_hbm.at[idx_ring.at[i % NBUFI, 0]],
                         rows_v.at[b], sem_g.at[b])

    def wait_gather(i, b):
        pltpu.make_async_copy(x_hbm.at[idx_ring.at[i % NBUFI, 0]],
                              rows_v.at[b], sem_g.at[b]).wait()

    def start_scatter(i, b):
        pltpu.async_copy(rows_v.at[b], agg_sp.at[idx_ring.at[i % NBUFI, 1]],
                         sem_s.at[b], add=True)
        pltpu.async_copy(ones_v, cnt_sp.at[idx_ring.at[i % NBUFI, 1]],
                         sem_c.at[b], add=True)

    def wait_scatter(i, b):
        pltpu.make_async_copy(rows_v.at[b],
                              agg_sp.at[idx_ring.at[i % NBUFI, 1]],
                              sem_s.at[b]).wait()
        pltpu.make_async_copy(ones_v, cnt_sp.at[idx_ring.at[i % NBUFI, 1]],
                              sem_c.at[b]).wait()

    for i in range(4):
        start_idx(i, i)

    def zrow(r, _):
        def zlane(k, __):
            rows_v[0, r, pl.ds(k * 16, 16)] = jnp.zeros((16,), jnp.float32)
            return 0
        lax.fori_loop(0, D // 16, zlane, 0)
        zb_v[r, pl.ds(0, 16)] = jnp.zeros((16,), jnp.float32)
        ones_v[r, pl.ds(0, 16)] = jnp.ones((16,), jnp.float32)
        return 0
    lax.fori_loop(0, CHUNK, zrow, 0)

    base_r = s * ROWS_PER_SUBCORE
    n_full = ROWS_PER_SUBCORE // CHUNK
    rem = ROWS_PER_SUBCORE - n_full * CHUNK

    def zcp(j, _):
        pltpu.sync_copy(rows_v.at[0],
                        agg_sp.at[pl.ds(base_r + j * CHUNK, CHUNK)])
        pltpu.sync_copy(zb_v, cnt_sp.at[pl.ds(base_r + j * CHUNK, CHUNK)])
        return 0
    lax.fori_loop(0, n_full, zcp, 0)
    pltpu.sync_copy(rows_v.at[0, pl.ds(0, rem)],
                    agg_sp.at[pl.ds(base_r + n_full * CHUNK, rem)])
    pltpu.sync_copy(zb_v.at[pl.ds(0, rem)],
                    cnt_sp.at[pl.ds(base_r + n_full * CHUNK, rem)])

    plsc.subcore_barrier()

    for i in range(2):
        wait_idx(i, i)
        start_gather(i, i % NBUF)

    def step(i, b):
        wait_gather(i, b)
        start_scatter(i, b)

        @pl.when(i + 2 < NUM_CHUNKS)
        def _():
            bb = (b + 2) % NBUF

            @pl.when(i >= NBUF - 2)
            def _():
                wait_scatter(i + 2 - NBUF, bb)
            wait_idx(i + 2, (i + 2) % NBUFI)
            start_gather(i + 2, bb)

        @pl.when(i + 4 < NUM_CHUNKS)
        def _():
            start_idx(i + 4, (i + 4) % NBUFI)

    def outer(j, _):
        for b in range(NBUF):
            step(j * NBUF + b, b)
        return 0
    n_main = NUM_CHUNKS // NBUF
    lax.fori_loop(0, n_main, outer, 0)

    for i in range(n_main * NBUF, NUM_CHUNKS):
        step(i, i % NBUF)
    for i in range(NUM_CHUNKS - NBUF, NUM_CHUNKS):
        wait_scatter(i, i % NBUF)

    plsc.subcore_barrier()

    pltpu.sync_copy(agg_sp.at[pl.ds(base_r, ROWS_PER_SUBCORE)],
                    agg_hbm.at[c, pl.ds(base_r, ROWS_PER_SUBCORE)])
    pltpu.sync_copy(cnt_sp.at[pl.ds(base_r, ROWS_PER_SUBCORE)],
                    cnt_hbm.at[c, pl.ds(base_r, ROWS_PER_SUBCORE)])


@functools.partial(
    pl.kernel,
    mesh=plsc.VectorSubcoreMesh(core_axis_name="c", subcore_axis_name="s"),
    out_type=[
        jax.ShapeDtypeStruct((NUM_CORES, N_NODES, D), jnp.float32),
        jax.ShapeDtypeStruct((NUM_CORES, N_NODES, CW), jnp.float32),
    ],
    scratch_types=[
        pltpu.VMEM_SHARED((N_NODES, D), jnp.float32),
        pltpu.VMEM_SHARED((N_NODES, CW), jnp.float32),
        pltpu.VMEM((NBUFI, 2, CHUNK), jnp.int32),
        pltpu.VMEM((NBUF, CHUNK, D), jnp.float32),
        pltpu.VMEM((CHUNK, CW), jnp.float32),
        pltpu.VMEM((CHUNK, CW), jnp.float32),
        pltpu.SemaphoreType.DMA((NBUF,)),
        pltpu.SemaphoreType.DMA((NBUF,)),
        pltpu.SemaphoreType.DMA((NBUF,)),
        pltpu.SemaphoreType.DMA((NBUFI,)),
    ],
    compiler_params=pltpu.CompilerParams(use_tc_tiling_on_sc=False,
                                         skip_device_barrier=True),
)
def _sc_agg(x_hbm, eidx_hbm, agg_hbm, cnt_hbm, agg_sp, cnt_sp, idx_ring,
            rows_v, ones_v, zb_v, sem_g, sem_s, sem_c, sem_x):
    _sc_agg_kernel(x_hbm, eidx_hbm, agg_hbm, cnt_hbm, agg_sp, cnt_sp,
                   idx_ring, rows_v, ones_v, zb_v, sem_g, sem_s, sem_c,
                   sem_x)


ROW_BLOCK = 1000


def _tc_dense_kernel(agg_ref, cnt_ref, x_ref, wl_ref, wr_ref, b_ref, g_ref,
                     be_ref, o_ref):
    feats = agg_ref[0] + agg_ref[1]
    cnt = (cnt_ref[0] + cnt_ref[1])[:, :1]
    mean = feats / jnp.maximum(cnt, 1.0)
    x = x_ref[...]
    h = (jnp.dot(mean, wl_ref[...], preferred_element_type=jnp.float32)
         + jnp.dot(x, wr_ref[...], preferred_element_type=jnp.float32)
         + b_ref[...])
    mu = jnp.mean(h, axis=1, keepdims=True)
    var = jnp.mean((h - mu) ** 2, axis=1, keepdims=True)
    h = (h - mu) / jnp.sqrt(var + EPS) * g_ref[...] + be_ref[...]
    o_ref[...] = jnp.maximum(h, 0.0) + x


def _tc_dense(agg2, cnt2, x, W_l, W_r, b, gamma, beta):
    grid = (N_NODES // ROW_BLOCK,)
    return pl.pallas_call(
        _tc_dense_kernel,
        grid=grid,
        in_specs=[
            pl.BlockSpec((NUM_CORES, ROW_BLOCK, D), lambda i: (0, i, 0)),
            pl.BlockSpec((NUM_CORES, ROW_BLOCK, CW), lambda i: (0, i, 0)),
            pl.BlockSpec((ROW_BLOCK, D), lambda i: (i, 0)),
            pl.BlockSpec((D, D), lambda i: (0, 0)),
            pl.BlockSpec((D, D), lambda i: (0, 0)),
            pl.BlockSpec((1, D), lambda i: (0, 0)),
            pl.BlockSpec((1, D), lambda i: (0, 0)),
            pl.BlockSpec((1, D), lambda i: (0, 0)),
        ],
        out_specs=pl.BlockSpec((ROW_BLOCK, D), lambda i: (i, 0)),
        out_shape=jax.ShapeDtypeStruct((N_NODES, D), jnp.float32),
    )(agg2, cnt2, x, W_l, W_r, b, gamma, beta)


def kernel(x, edge_index, W_l, b_l, W_r, b_r, gamma, beta):
    eidx = edge_index.astype(jnp.int32)
    agg2, cnt2 = _sc_agg(x, eidx)
    b = (b_l + b_r).reshape(1, D)
    return _tc_dense(agg2, cnt2, x, W_l, W_r, b, gamma.reshape(1, D),
                     beta.reshape(1, D))


# speedup vs baseline: 1.2763x; 1.2763x over previous
import functools

import jax
import jax.numpy as jnp
from jax import lax
from jax.experimental import pallas as pl
from jax.experimental.pallas import tpu as pltpu
from jax.experimental.pallas import tpu_sc as plsc

N_NODES = 10000
N_EDGES = 320000
D = 128
CW = 16
EPS = 1e-5

NUM_CORES = 2
NUM_SUBCORES = 16
NUM_WORKERS = NUM_CORES * NUM_SUBCORES
EDGES_PER_WORKER = N_EDGES // NUM_WORKERS
CHUNK = 80
NUM_CHUNKS = EDGES_PER_WORKER // CHUNK
NBUF = 3
NBUFI = 5
ROWS_PER_SUBCORE = N_NODES // NUM_SUBCORES


def _sc_agg_kernel(x_hbm, eidx_hbm, agg_hbm, cnt_hbm, agg_sp, cnt_sp,
                   idx_ring, rows_v, ones_v, zb_v, sem_g, sem_s, sem_c,
                   sem_x):
    c = lax.axis_index("c")
    s = lax.axis_index("s")
    wid = s * NUM_CORES + c
    chunk_base = wid * NUM_CHUNKS

    def start_idx(i, bi):
        e0 = (chunk_base + i) * CHUNK
        pltpu.async_copy(eidx_hbm.at[0, pl.ds(e0, CHUNK)], idx_ring.at[bi, 0],
                         sem_x.at[bi])
        pltpu.async_copy(eidx_hbm.at[1, pl.ds(e0, CHUNK)], idx_ring.at[bi, 1],
                         sem_x.at[bi])

    def wait_idx(i, bi):
        e0 = (chunk_base + i) * CHUNK
        pltpu.make_async_copy(eidx_hbm.at[0, pl.ds(e0, CHUNK)],
                              idx_ring.at[bi, 0], sem_x.at[bi]).wait()
        pltpu.make_async_copy(eidx_hbm.at[1, pl.ds(e0, CHUNK)],
                              idx_ring.at[bi, 1], sem_x.at[bi]).wait()

    def start_gather(i, b):
        pltpu.async_copy(x_hbm.at[idx_ring.at[i % NBUFI, 0]],
                         rows_v.at[b], sem_g.at[b])

    def wait_gather(i, b):
        pltpu.make_async_copy(x_hbm.at[idx_ring.at[i % NBUFI, 0]],
                              rows_v.at[b], sem_g.at[b]).wait()

    def start_scatter(i, b):
        pltpu.async_copy(rows_v.at[b], agg_sp.at[idx_ring.at[i % NBUFI, 1]],
                         sem_s.at[b], add=True)
        pltpu.async_copy(ones_v, cnt_sp.at[idx_ring.at[i % NBUFI, 1]],
                         sem_c.at[b], add=True)

    def wait_scatter(i, b):
        pltpu.make_async_copy(rows_v.at[b],
                              agg_sp.at[idx_ring.at[i % NBUFI, 1]],
                              sem_s.at[b]).wait()
        pltpu.make_async_copy(ones_v, cnt_sp.at[idx_ring.at[i % NBUFI, 1]],
                              sem_c.at[b]).wait()

    for i in range(4):
        start_idx(i, i)

    def zrow(r, _):
        def zlane(k, __):
            rows_v[2, r, pl.ds(k * 16, 16)] = jnp.zeros((16,), jnp.float32)
            return 0
        lax.fori_loop(0, D // 16, zlane, 0)
        zb_v[r, pl.ds(0, 16)] = jnp.zeros((16,), jnp.float32)
        ones_v[r, pl.ds(0, 16)] = jnp.ones((16,), jnp.float32)
        return 0
    lax.fori_loop(0, CHUNK, zrow, 0)

    for i in range(2):
        wait_idx(i, i)
        start_gather(i, i % NBUF)

    base_r = s * ROWS_PER_SUBCORE
    n_full = ROWS_PER_SUBCORE // CHUNK
    rem = ROWS_PER_SUBCORE - n_full * CHUNK

    def zcp(j, _):
        pltpu.sync_copy(rows_v.at[2],
                        agg_sp.at[pl.ds(base_r + j * CHUNK, CHUNK)])
        pltpu.sync_copy(zb_v, cnt_sp.at[pl.ds(base_r + j * CHUNK, CHUNK)])
        return 0
    lax.fori_loop(0, n_full, zcp, 0)
    pltpu.sync_copy(rows_v.at[2, pl.ds(0, rem)],
                    agg_sp.at[pl.ds(base_r + n_full * CHUNK, rem)])
    pltpu.sync_copy(zb_v.at[pl.ds(0, rem)],
                    cnt_sp.at[pl.ds(base_r + n_full * CHUNK, rem)])

    plsc.subcore_barrier()

    def step(i, b):
        wait_gather(i, b)
        start_scatter(i, b)

        @pl.when(i + 2 < NUM_CHUNKS)
        def _():
            bb = (b + 2) % NBUF

            @pl.when(i >= NBUF - 2)
            def _():
                wait_scatter(i + 2 - NBUF, bb)
            wait_idx(i + 2, (i + 2) % NBUFI)
            start_gather(i + 2, bb)

        @pl.when(i + 4 < NUM_CHUNKS)
        def _():
            start_idx(i + 4, (i + 4) % NBUFI)

    def outer(j, _):
        for b in range(NBUF):
            step(j * NBUF + b, b)
        return 0
    n_main = NUM_CHUNKS // NBUF
    lax.fori_loop(0, n_main, outer, 0)

    for i in range(n_main * NBUF, NUM_CHUNKS):
        step(i, i % NBUF)
    for i in range(NUM_CHUNKS - NBUF, NUM_CHUNKS):
        wait_scatter(i, i % NBUF)

    plsc.subcore_barrier()

    pltpu.sync_copy(agg_sp.at[pl.ds(base_r, ROWS_PER_SUBCORE)],
                    agg_hbm.at[c, pl.ds(base_r, ROWS_PER_SUBCORE)])
    pltpu.sync_copy(cnt_sp.at[pl.ds(base_r, ROWS_PER_SUBCORE)],
                    cnt_hbm.at[c, pl.ds(base_r, ROWS_PER_SUBCORE)])


@functools.partial(
    pl.kernel,
    mesh=plsc.VectorSubcoreMesh(core_axis_name="c", subcore_axis_name="s"),
    out_type=[
        jax.ShapeDtypeStruct((NUM_CORES, N_NODES, D), jnp.float32),
        jax.ShapeDtypeStruct((NUM_CORES, N_NODES, CW), jnp.float32),
    ],
    scratch_types=[
        pltpu.VMEM_SHARED((N_NODES, D), jnp.float32),
        pltpu.VMEM_SHARED((N_NODES, CW), jnp.float32),
        pltpu.VMEM((NBUFI, 2, CHUNK), jnp.int32),
        pltpu.VMEM((NBUF, CHUNK, D), jnp.float32),
        pltpu.VMEM((CHUNK, CW), jnp.float32),
        pltpu.VMEM((CHUNK, CW), jnp.float32),
        pltpu.SemaphoreType.DMA((NBUF,)),
        pltpu.SemaphoreType.DMA((NBUF,)),
        pltpu.SemaphoreType.DMA((NBUF,)),
        pltpu.SemaphoreType.DMA((NBUFI,)),
    ],
    compiler_params=pltpu.CompilerParams(use_tc_tiling_on_sc=False,
                                         skip_device_barrier=True),
)
def _sc_agg(x_hbm, eidx_hbm, agg_hbm, cnt_hbm, agg_sp, cnt_sp, idx_ring,
            rows_v, ones_v, zb_v, sem_g, sem_s, sem_c, sem_x):
    _sc_agg_kernel(x_hbm, eidx_hbm, agg_hbm, cnt_hbm, agg_sp, cnt_sp,
                   idx_ring, rows_v, ones_v, zb_v, sem_g, sem_s, sem_c,
                   sem_x)


ROW_BLOCK = 1000


def _tc_dense_kernel(agg_ref, cnt_ref, x_ref, wl_ref, wr_ref, b_ref, g_ref,
                     be_ref, o_ref):
    feats = agg_ref[0] + agg_ref[1]
    cnt = (cnt_ref[0] + cnt_ref[1])[:, :1]
    mean = feats / jnp.maximum(cnt, 1.0)
    x = x_ref[...]
    h = (jnp.dot(mean, wl_ref[...], preferred_element_type=jnp.float32)
         + jnp.dot(x, wr_ref[...], preferred_element_type=jnp.float32)
         + b_ref[...])
    mu = jnp.mean(h, axis=1, keepdims=True)
    var = jnp.mean((h - mu) ** 2, axis=1, keepdims=True)
    h = (h - mu) / jnp.sqrt(var + EPS) * g_ref[...] + be_ref[...]
    o_ref[...] = jnp.maximum(h, 0.0) + x


def _tc_dense(agg2, cnt2, x, W_l, W_r, b, gamma, beta):
    grid = (N_NODES // ROW_BLOCK,)
    return pl.pallas_call(
        _tc_dense_kernel,
        grid=grid,
        in_specs=[
            pl.BlockSpec((NUM_CORES, ROW_BLOCK, D), lambda i: (0, i, 0)),
            pl.BlockSpec((NUM_CORES, ROW_BLOCK, CW), lambda i: (0, i, 0)),
            pl.BlockSpec((ROW_BLOCK, D), lambda i: (i, 0)),
            pl.BlockSpec((D, D), lambda i: (0, 0)),
            pl.BlockSpec((D, D), lambda i: (0, 0)),
            pl.BlockSpec((1, D), lambda i: (0, 0)),
            pl.BlockSpec((1, D), lambda i: (0, 0)),
            pl.BlockSpec((1, D), lambda i: (0, 0)),
        ],
        out_specs=pl.BlockSpec((ROW_BLOCK, D), lambda i: (i, 0)),
        out_shape=jax.ShapeDtypeStruct((N_NODES, D), jnp.float32),
    )(agg2, cnt2, x, W_l, W_r, b, gamma, beta)


def kernel(x, edge_index, W_l, b_l, W_r, b_r, gamma, beta):
    eidx = edge_index.astype(jnp.int32)
    agg2, cnt2 = _sc_agg(x, eidx)
    b = (b_l + b_r).reshape(1, D)
    return _tc_dense(agg2, cnt2, x, W_l, W_r, b, gamma.reshape(1, D),
                     beta.reshape(1, D))
